# baseline (speedup 1.0000x reference)
import functools

import jax
import jax.numpy as jnp
from jax import lax
from jax.experimental import pallas as pl
from jax.experimental.pallas import tpu as pltpu
from jax.experimental.pallas import tpu_sc as plsc

_G = 4


def kernel(x, weight):
    B, H = x.shape
    V, D = weight.shape
    info = plsc.get_sparse_core_info()
    nw = info.num_cores * info.num_subcores
    rows_w = B // nw
    ng = rows_w // _G
    assert B == nw * ng * _G and ng % 2 == 0, (B, nw, ng)

    mesh = plsc.VectorSubcoreMesh(core_axis_name="c", subcore_axis_name="s")

    @functools.partial(
        pl.kernel,
        mesh=mesh,
        out_type=jax.ShapeDtypeStruct((B, H, D), jnp.float32),
        scratch_types=[
            pltpu.VMEM((rows_w, H), jnp.int32),
            pltpu.VMEM((2, _G, H, D), jnp.float32),
            pltpu.SemaphoreType.DMA,
            pltpu.SemaphoreType.DMA,
            pltpu.SemaphoreType.DMA,
            pltpu.SemaphoreType.DMA,
        ],
        compiler_params=pltpu.CompilerParams(use_tc_tiling_on_sc=False),
    )
    def run(x_hbm, w_hbm, out_hbm, idx_v, rows_v, gsem0, gsem1, osem0, osem1):
        wid = lax.axis_index("s") * info.num_cores + lax.axis_index("c")
        base = wid * rows_w
        gsems = (gsem0, gsem1)
        osems = (osem0, osem1)
        pltpu.sync_copy(x_hbm.at[pl.ds(base, rows_w)], idx_v)

        def issue_gathers(g, bank):
            for j in range(_G):
                r = g * _G + j
                pltpu.async_copy(
                    w_hbm.at[idx_v.at[r]],
                    rows_v.at[bank].at[j],
                    gsems[bank],
                )

        def wait_gathers(bank):
            pltpu.make_async_copy(
                out_hbm.at[pl.ds(0, _G)], rows_v.at[bank], gsems[bank]
            ).wait()

        def issue_out(g, bank):
            pltpu.async_copy(
                rows_v.at[bank],
                out_hbm.at[pl.ds(base + g * _G, _G)],
                osems[bank],
            )

        def wait_out(bank):
            pltpu.make_async_copy(
                rows_v.at[bank], out_hbm.at[pl.ds(base, _G)], osems[bank]
            ).wait()

        issue_gathers(0, 0)
        issue_gathers(1, 1)
        wait_gathers(0)
        issue_out(0, 0)

        def pair(p, carry):
            for b, g in ((1, 2 * p + 1), (0, 2 * p + 2)):
                wait_out(1 - b)
                issue_gathers(g + 1, 1 - b)
                wait_gathers(b)
                issue_out(g, b)
            return carry

        lax.fori_loop(0, (ng - 2) // 2, pair, 0)

        wait_out(0)
        wait_gathers(1)
        issue_out(ng - 1, 1)
        wait_out(1)

    return run(x.astype(jnp.int32), weight)
